# Initial kernel scaffold; baseline (speedup 1.0000x reference)
import jax
import jax.numpy as jnp
from jax.experimental import pallas as pl


def kernel(attn_edge_type, edge_weight, virtual_weight):
    raise NotImplementedError("write your pallas kernel here")


# speedup vs baseline: 4.1251x; 4.1251x over previous
import functools

import jax
import jax.numpy as jnp
from jax import lax
from jax.experimental import pallas as pl
from jax.experimental.pallas import tpu as pltpu
from jax.experimental.pallas import tpu_sc as plsc

NUM_EDGES = 1536
HIDDEN = 32
BS = 8
N_NODE = 256
EDGE_FEAT = 3
L = 16
NC, NS = 2, 16
NW = NC * NS
N_OUT = N_NODE + 1
TASKS = BS * N_NODE
TASKS_PER_W = TASKS // NW
ROWS_PER_W = N_NODE // (NW // BS)
BLK_W = 272
N_CHUNK = 17


def _body(idx_hbm, tab_hbm, vw_hbm, out_hbm, table_v, idx_v, block_v, vw_v):
    wid = lax.axis_index("s") * NC + lax.axis_index("c")
    b = wid // 4
    i0 = (wid % 4) * ROWS_PER_W

    pltpu.sync_copy(tab_hbm, table_v)
    pltpu.sync_copy(vw_hbm, vw_v)

    iota = lax.iota(jnp.int32, L)
    third = jnp.full((L,), 1.0 / 3.0, jnp.float32)
    mask0 = iota >= 1
    maskL = iota == 0
    lastc = jnp.minimum(16 * (N_CHUNK - 1) + iota, N_OUT - 1)

    def fill_h(h, _):
        hs = jnp.full((L,), h, jnp.int32)
        vwh = plsc.load_gather(vw_v, [hs])
        for k in range(N_CHUNK - 1):
            block_v[h, pl.ds(16 * k, 16)] = vwh
        plsc.store_scatter(block_v, [hs, lastc], vwh, mask=maskL)
        return 0

    lax.fori_loop(0, HIDDEN, fill_h, 0, unroll=False)

    @pl.when(wid % 4 == 0)
    def _():
        pltpu.sync_copy(block_v, out_hbm.at[b, :, 0, :])

    def task_body(t, _):
        i = i0 + t
        pltpu.sync_copy(idx_hbm.at[pl.ds((b * N_NODE + i) * N_NODE * EDGE_FEAT,
                                         N_NODE * EDGE_FEAT)], idx_v)
        for k in range(N_CHUNK):
            jj = jnp.clip(16 * k + iota - 1, 0, N_NODE - 1)
            ci = jj * EDGE_FEAT
            a0 = plsc.load_gather(idx_v, [ci]) * HIDDEN
            a1 = plsc.load_gather(idx_v, [ci + 1]) * HIDDEN
            a2 = plsc.load_gather(idx_v, [ci + 2]) * HIDDEN

            def h_body(h, _):
                hs = jnp.full((L,), h, jnp.int32)
                g0 = plsc.load_gather(table_v, [a0 + hs])
                g1 = plsc.load_gather(table_v, [a1 + hs])
                g2 = plsc.load_gather(table_v, [a2 + hs])
                val = (g0 + g1 + g2) * third
                if k == 0:
                    plsc.store_scatter(block_v, [hs, iota], val, mask=mask0)
                elif k == N_CHUNK - 1:
                    plsc.store_scatter(block_v, [hs, lastc], val, mask=maskL)
                else:
                    block_v[h, pl.ds(16 * k, 16)] = val
                return 0

            lax.fori_loop(0, HIDDEN, h_body, 0, unroll=False)

        pltpu.sync_copy(block_v, out_hbm.at[b, :, i + 1, :])
        return 0

    lax.fori_loop(0, TASKS_PER_W, task_body, 0, unroll=False)


@jax.jit
def _edge_feature_sc(idx_flat, tab_flat, vw_flat):
    mesh = plsc.VectorSubcoreMesh(core_axis_name="c", subcore_axis_name="s")
    return pl.kernel(
        _body,
        out_type=jax.ShapeDtypeStruct((BS, HIDDEN, N_OUT, N_OUT), jnp.float32),
        mesh=mesh,
        compiler_params=pltpu.CompilerParams(use_tc_tiling_on_sc=False,
                                             needs_layout_passes=False),
        scratch_types=[
            pltpu.VMEM((NUM_EDGES * HIDDEN,), jnp.float32),
            pltpu.VMEM((N_NODE * EDGE_FEAT,), jnp.int32),
            pltpu.VMEM((HIDDEN, N_OUT), jnp.float32),
            pltpu.VMEM((HIDDEN,), jnp.float32),
        ],
    )(idx_flat, tab_flat, vw_flat)


def kernel(attn_edge_type, edge_weight, virtual_weight):
    idx_flat = attn_edge_type.reshape(-1)
    tab_flat = edge_weight.reshape(-1)
    vw_flat = virtual_weight.reshape(-1)
    return _edge_feature_sc(idx_flat, tab_flat, vw_flat)


# speedup vs baseline: 6.3980x; 1.5510x over previous
import functools

import jax
import jax.numpy as jnp
from jax import lax
from jax.experimental import pallas as pl
from jax.experimental.pallas import tpu as pltpu
from jax.experimental.pallas import tpu_sc as plsc

NUM_EDGES = 1536
HIDDEN = 32
HPAIRS = HIDDEN // 2
BS = 8
N_NODE = 256
EDGE_FEAT = 3
ROW_I = N_NODE * EDGE_FEAT
L = 16
NC, NS = 2, 16
NW = NC * NS
N_OUT = N_NODE + 1
TASKS = BS * N_NODE
TASKS_PER_W = TASKS // NW
W_PER_B = NW // BS
ROWS_PER_W = N_NODE // W_PER_B
N_CHUNK = 17


def _body(idx_hbm, tab_hbm, vw_hbm, out_hbm,
          table_v, idx_v, blocks, vw_v, sem_idx, sem_out):
    wid = lax.axis_index("s") * NC + lax.axis_index("c")
    b = wid // W_PER_B
    i0 = (wid % W_PER_B) * ROWS_PER_W

    pltpu.sync_copy(tab_hbm, table_v)
    pltpu.sync_copy(vw_hbm, vw_v)

    iota = lax.iota(jnp.int32, L)
    mask0 = iota >= 1
    maskL = iota == 0
    lastc = jnp.minimum(16 * (N_CHUNK - 1) + iota, N_OUT - 1)
    himask = jnp.full((L,), -65536, jnp.int32)

    def fill_h(h, _):
        hs = jnp.full((L,), h, jnp.int32)
        vwh = plsc.load_gather(vw_v, [hs])
        for blk in blocks:
            for k in range(N_CHUNK - 1):
                blk[h, pl.ds(16 * k, 16)] = vwh
            plsc.store_scatter(blk, [hs, lastc], vwh, mask=maskL)
        return 0

    lax.fori_loop(0, HIDDEN, fill_h, 0, unroll=False)

    @pl.when(wid % W_PER_B == 0)
    def _():
        pltpu.sync_copy(blocks[0], out_hbm.at[b, :, 0, :])

    def idx_dma(t, p):
        return pltpu.make_async_copy(
            idx_hbm.at[pl.ds((b * N_NODE + i0 + t) * ROW_I, ROW_I)],
            idx_v.at[pl.ds(p * ROW_I, ROW_I)], sem_idx[p])

    def out_dma(t, p):
        return pltpu.make_async_copy(
            blocks[p], out_hbm.at[b, :, i0 + t + 1, :], sem_out[p])

    def compute(t, p):
        blk = blocks[p]
        base = p * ROW_I
        for k in range(N_CHUNK):
            jj = jnp.clip(16 * k + iota - 1, 0, N_NODE - 1)
            ci = base + jj * EDGE_FEAT
            a0 = plsc.load_gather(idx_v, [ci]) * HPAIRS
            a1 = plsc.load_gather(idx_v, [ci + 1]) * HPAIRS
            a2 = plsc.load_gather(idx_v, [ci + 2]) * HPAIRS

            def h_body(hp, _):
                hs = jnp.full((L,), hp, jnp.int32)
                g0 = plsc.load_gather(table_v, [a0 + hs])
                g1 = plsc.load_gather(table_v, [a1 + hs])
                g2 = plsc.load_gather(table_v, [a2 + hs])
                s = (plsc.bitcast(g0, jnp.bfloat16)
                     + plsc.bitcast(g1, jnp.bfloat16)
                     + plsc.bitcast(g2, jnp.bfloat16))
                sw = plsc.bitcast(s, jnp.int32)
                even = plsc.bitcast(lax.shift_left(sw, 16), jnp.float32)
                odd = plsc.bitcast(lax.bitwise_and(sw, himask), jnp.float32)
                h2 = hp + hp
                if k == 0:
                    plsc.store_scatter(blk, [hs + hs, iota], even, mask=mask0)
                    plsc.store_scatter(blk, [hs + hs + 1, iota], odd, mask=mask0)
                elif k == N_CHUNK - 1:
                    plsc.store_scatter(blk, [hs + hs, lastc], even, mask=maskL)
                    plsc.store_scatter(blk, [hs + hs + 1, lastc], odd, mask=maskL)
                else:
                    blk[h2, pl.ds(16 * k, 16)] = even
                    blk[h2 + 1, pl.ds(16 * k, 16)] = odd
                return 0

            lax.fori_loop(0, HPAIRS, h_body, 0, unroll=False)

    idx_dma(0, 0).start()

    def task_pair(t2, _):
        for p in (0, 1):
            t = t2 + t2 + p

            @pl.when(t + 1 < TASKS_PER_W)
            def _():
                idx_dma(t + 1, 1 - p).start()

            idx_dma(t, p).wait()

            @pl.when(t >= 2)
            def _():
                out_dma(t - 2, p).wait()

            compute(t, p)
            out_dma(t, p).start()
        return 0

    lax.fori_loop(0, TASKS_PER_W // 2, task_pair, 0, unroll=False)
    out_dma(TASKS_PER_W - 2, 0).wait()
    out_dma(TASKS_PER_W - 1, 1).wait()


@jax.jit
def _edge_feature_sc(idx_flat, tab_packed, vw_flat):
    mesh = plsc.VectorSubcoreMesh(core_axis_name="c", subcore_axis_name="s")
    return pl.kernel(
        _body,
        out_type=jax.ShapeDtypeStruct((BS, HIDDEN, N_OUT, N_OUT), jnp.float32),
        mesh=mesh,
        compiler_params=pltpu.CompilerParams(use_tc_tiling_on_sc=False,
                                             needs_layout_passes=False),
        scratch_types=[
            pltpu.VMEM((NUM_EDGES * HPAIRS,), jnp.int32),
            pltpu.VMEM((2 * ROW_I,), jnp.int32),
            [pltpu.VMEM((HIDDEN, N_OUT), jnp.float32) for _ in range(2)],
            pltpu.VMEM((HIDDEN,), jnp.float32),
            [pltpu.SemaphoreType.DMA for _ in range(2)],
            [pltpu.SemaphoreType.DMA for _ in range(2)],
        ],
    )(idx_flat, tab_packed, vw_flat)


def kernel(attn_edge_type, edge_weight, virtual_weight):
    idx_flat = attn_edge_type.reshape(-1)
    wb = lax.bitcast_convert_type(
        (edge_weight * (1.0 / 3.0)).astype(jnp.bfloat16), jnp.uint16)
    packed = lax.bitcast_convert_type(
        wb[:, 0::2].astype(jnp.uint32) | (wb[:, 1::2].astype(jnp.uint32) << 16),
        jnp.int32).reshape(-1)
    vw_flat = virtual_weight.reshape(-1)
    return _edge_feature_sc(idx_flat, packed, vw_flat)
